# baseline (speedup 1.0000x reference)
import functools
import math

import jax
import jax.numpy as jnp
from jax import lax
from jax.experimental import pallas as pl
from jax.experimental.pallas import tpu as pltpu
from jax.experimental.pallas import tpu_sc as plsc

_B = 2
_C = 32
_N = 1024
_KNN = 8
_NF = 15
_ROWS = _B * _C
_LANES = 16
_CHUNKS = _N // _LANES


def _conv2d(x, w, b, reflect=False):
    if reflect:
        x = jnp.pad(x, ((0, 0), (0, 0), (1, 1), (1, 1)), mode='reflect')
        pad = 'VALID'
    else:
        pad = ((1, 1), (1, 1))
    y = lax.conv_general_dilated(x, w, (1, 1), pad,
                                 dimension_numbers=('NCHW', 'OIHW', 'NCHW'))
    return y + b[None, :, None, None]


def _avgpool2(x):
    return lax.reduce_window(x, 0.0, lax.add, (1, 1, 2, 2), (1, 1, 2, 2), 'VALID') / 4.0


def _upsample2(x):
    return jnp.repeat(jnp.repeat(x, 2, axis=2), 2, axis=3)


def _uniform_grid(b, w, h):
    px = jnp.linspace(-1.0, 1.0, w, dtype=jnp.float32)
    py = jnp.linspace(-1.0, 1.0, h, dtype=jnp.float32)
    gx = jnp.broadcast_to(px[:, None], (w, h))
    gy = jnp.broadcast_to(py[None, :], (w, h))
    g = jnp.stack([gx, gy], axis=0)
    return jnp.broadcast_to(g[None], (b, 2, w, h))


def _sc_body(x_hbm, idx_hbm, out_hbm, idx_v, cur0, new0, cur1, new1):
    wid = lax.axis_index("s") * 2 + lax.axis_index("c")
    pltpu.sync_copy(idx_hbm, idx_v)
    rows = ((cur0, new0), (cur1, new1))
    for rloc in range(2):
        pltpu.sync_copy(x_hbm.at[2 * wid + rloc], rows[rloc][0])

    def step(i, carry):
        scale = jnp.where(i == 0, 128.0, 0.125).astype(jnp.float32)
        for rloc in range(2):
            cur, new = rows[rloc]

            def chunk(j, c2):
                b0 = pl.multiple_of(j * _LANES, _LANES)
                acc = plsc.load_gather(cur, [idx_v[0, pl.ds(b0, _LANES)]])
                for k in range(1, _KNN):
                    acc = acc + plsc.load_gather(cur, [idx_v[k, pl.ds(b0, _LANES)]])
                new[pl.ds(b0, _LANES)] = acc * scale
                return c2

            lax.fori_loop(0, _CHUNKS, chunk, None)
            r = 2 * wid + rloc
            pltpu.sync_copy(new, out_hbm.at[r // _C, i, r % _C])

            def copyback(j, c2):
                b0 = pl.multiple_of(j * _LANES, _LANES)
                cur[pl.ds(b0, _LANES)] = new[pl.ds(b0, _LANES)]
                return c2

            lax.fori_loop(0, _CHUNKS, copyback, None)
        return carry

    lax.fori_loop(0, _NF, step, None)


@jax.jit
def _sc_gather_steps(x_rows, knn_idx):
    mesh = plsc.VectorSubcoreMesh(core_axis_name="c", subcore_axis_name="s")
    f = pl.kernel(
        _sc_body,
        out_type=jax.ShapeDtypeStruct((_B, _NF, _C, _N), jnp.float32),
        mesh=mesh,
        scratch_types=[
            pltpu.VMEM((_KNN, _N), jnp.int32),
            pltpu.VMEM((_N,), jnp.float32),
            pltpu.VMEM((_N,), jnp.float32),
            pltpu.VMEM((_N,), jnp.float32),
            pltpu.VMEM((_N,), jnp.float32),
        ],
        compiler_params=pltpu.CompilerParams(needs_layout_passes=False),
    )
    return f(x_rows, knn_idx)


def kernel(x, conv1_w, conv1_b, ggd_w, ggd_b, conv2_w, conv2_b, vel_w1,
           vel_b1, vel_w2, vel_b2, dec1_w, dec1_b, ggu_w, ggu_b, dec3_w,
           dec3_b):
    x = x * 2.0 - 1.0
    first_frame = x[:, None]
    h = jax.nn.relu(_conv2d(x, conv1_w, conv1_b, reflect=True))
    h = jax.nn.relu(_conv2d(h, ggd_w, ggd_b, reflect=False))
    h = _avgpool2(h)
    h = jax.nn.relu(h)
    h = jax.nn.relu(_conv2d(h, conv2_w, conv2_b, reflect=True))
    h = _avgpool2(h)
    x_part = h.reshape(_B, _C, _N)

    pos = _uniform_grid(_B, 32, 32).reshape(_B, 2, _N)
    d = jnp.einsum('bci,bcj->bij', pos, pos)
    _, knn_ind = lax.top_k(-d, _KNN)
    idx = jnp.transpose(knn_ind, (0, 2, 1))[0]

    frames = jnp.broadcast_to(x_part[:, None] * 1024.0, (_B, _NF, _C, _N))
    y = frames.reshape(_B * _NF, _C, 32, 32)

    y = _upsample2(y)
    y = jax.nn.relu(_conv2d(y, dec1_w, dec1_b, reflect=True))
    y = jax.nn.relu(_conv2d(y, ggu_w, ggu_b, reflect=False))
    y = _upsample2(y)
    y = jax.nn.relu(y)
    y = jnp.tanh(_conv2d(y, dec3_w, dec3_b, reflect=True))
    y = y.reshape(_B, _NF, 3, 128, 128)
    y = jnp.concatenate([first_frame, y], axis=1)
    return (y + 1.0) / 2.0


# speedup vs baseline: 1.0676x; 1.0676x over previous
import functools
import math

import jax
import jax.numpy as jnp
from jax import lax
from jax.experimental import pallas as pl
from jax.experimental.pallas import tpu as pltpu
from jax.experimental.pallas import tpu_sc as plsc

_B = 2
_C = 32
_N = 1024
_KNN = 8
_NF = 15
_ROWS = _B * _C
_LANES = 16
_CHUNKS = _N // _LANES


def _conv2d_nhwc(x, w, b, reflect=False):
    if reflect:
        x = jnp.pad(x, ((0, 0), (1, 1), (1, 1), (0, 0)), mode='reflect')
        pad = 'VALID'
    else:
        pad = ((1, 1), (1, 1))
    y = lax.conv_general_dilated(x, jnp.transpose(w, (2, 3, 1, 0)), (1, 1), pad,
                                 dimension_numbers=('NHWC', 'HWIO', 'NHWC'))
    return y + b[None, None, None, :]


def _avgpool2_nhwc(x):
    return lax.reduce_window(x, 0.0, lax.add, (1, 2, 2, 1), (1, 2, 2, 1), 'VALID') / 4.0


def _upsample2_nhwc(x):
    return jnp.repeat(jnp.repeat(x, 2, axis=1), 2, axis=2)


def _uniform_grid(b, w, h):
    px = jnp.linspace(-1.0, 1.0, w, dtype=jnp.float32)
    py = jnp.linspace(-1.0, 1.0, h, dtype=jnp.float32)
    gx = jnp.broadcast_to(px[:, None], (w, h))
    gy = jnp.broadcast_to(py[None, :], (w, h))
    g = jnp.stack([gx, gy], axis=0)
    return jnp.broadcast_to(g[None], (b, 2, w, h))


def _sc_body(x_hbm, idx_hbm, out_hbm, idx_v, cur0, new0, cur1, new1):
    wid = lax.axis_index("s") * 2 + lax.axis_index("c")
    pltpu.sync_copy(idx_hbm, idx_v)
    rows = ((cur0, new0), (cur1, new1))
    for rloc in range(2):
        pltpu.sync_copy(x_hbm.at[2 * wid + rloc], rows[rloc][0])

    def step(i, carry):
        scale = jnp.where(i == 0, 128.0, 0.125).astype(jnp.float32)
        for rloc in range(2):
            cur, new = rows[rloc]

            def chunk(j, c2):
                b0 = pl.multiple_of(j * _LANES, _LANES)
                acc = plsc.load_gather(cur, [idx_v[0, pl.ds(b0, _LANES)]])
                for k in range(1, _KNN):
                    acc = acc + plsc.load_gather(cur, [idx_v[k, pl.ds(b0, _LANES)]])
                new[pl.ds(b0, _LANES)] = acc * scale
                return c2

            lax.fori_loop(0, _CHUNKS, chunk, None)
            r = 2 * wid + rloc
            pltpu.sync_copy(new, out_hbm.at[r // _C, i, r % _C])

            def copyback(j, c2):
                b0 = pl.multiple_of(j * _LANES, _LANES)
                cur[pl.ds(b0, _LANES)] = new[pl.ds(b0, _LANES)]
                return c2

            lax.fori_loop(0, _CHUNKS, copyback, None)
        return carry

    lax.fori_loop(0, _NF, step, None)


@jax.jit
def _sc_gather_steps(x_rows, knn_idx):
    mesh = plsc.VectorSubcoreMesh(core_axis_name="c", subcore_axis_name="s")
    f = pl.kernel(
        _sc_body,
        out_type=jax.ShapeDtypeStruct((_B, _NF, _C, _N), jnp.float32),
        mesh=mesh,
        scratch_types=[
            pltpu.VMEM((_KNN, _N), jnp.int32),
            pltpu.VMEM((_N,), jnp.float32),
            pltpu.VMEM((_N,), jnp.float32),
            pltpu.VMEM((_N,), jnp.float32),
            pltpu.VMEM((_N,), jnp.float32),
        ],
        compiler_params=pltpu.CompilerParams(needs_layout_passes=False),
    )
    return f(x_rows, knn_idx)


def kernel(x, conv1_w, conv1_b, ggd_w, ggd_b, conv2_w, conv2_b, vel_w1,
           vel_b1, vel_w2, vel_b2, dec1_w, dec1_b, ggu_w, ggu_b, dec3_w,
           dec3_b):
    x = x * 2.0 - 1.0
    first_frame = x[:, None]
    xh = jnp.transpose(x, (0, 2, 3, 1))
    h = jax.nn.relu(_conv2d_nhwc(xh, conv1_w, conv1_b, reflect=True))
    h = jax.nn.relu(_conv2d_nhwc(h, ggd_w, ggd_b, reflect=False))
    h = _avgpool2_nhwc(h)
    h = jax.nn.relu(h)
    h = jax.nn.relu(_conv2d_nhwc(h, conv2_w, conv2_b, reflect=True))
    h = _avgpool2_nhwc(h)
    x_part = jnp.transpose(h.reshape(_B, _N, _C), (0, 2, 1))

    pos = _uniform_grid(_B, 32, 32).reshape(_B, 2, _N)
    d = jnp.einsum('bci,bcj->bij', pos, pos)
    _, knn_ind = lax.top_k(-d, _KNN)
    idx = jnp.transpose(knn_ind, (0, 2, 1))[0]

    frames = _sc_gather_steps(x_part.reshape(_ROWS, _N), idx)
    y = jnp.transpose(frames, (0, 1, 3, 2)).reshape(_B * _NF, 32, 32, _C)

    y = _upsample2_nhwc(y)
    y = jax.nn.relu(_conv2d_nhwc(y, dec1_w, dec1_b, reflect=True))
    y = jax.nn.relu(_conv2d_nhwc(y, ggu_w, ggu_b, reflect=False))
    y = _upsample2_nhwc(y)
    y = jnp.tanh(_conv2d_nhwc(y, dec3_w, dec3_b, reflect=True))
    y = jnp.transpose(y, (0, 3, 1, 2)).reshape(_B, _NF, 3, 128, 128)
    y = jnp.concatenate([first_frame, y], axis=1)
    return (y + 1.0) / 2.0


# speedup vs baseline: 1.0783x; 1.0100x over previous
import functools
import math

import jax
import jax.numpy as jnp
from jax import lax
from jax.experimental import pallas as pl
from jax.experimental.pallas import tpu as pltpu
from jax.experimental.pallas import tpu_sc as plsc

_B = 2
_C = 32
_N = 1024
_KNN = 8
_NF = 15
_ROWS = _B * _C
_LANES = 16
_CHUNKS = _N // _LANES


def _conv2d_nhwc(x, w, b, reflect=False):
    if reflect:
        x = jnp.pad(x, ((0, 0), (1, 1), (1, 1), (0, 0)), mode='reflect')
        pad = 'VALID'
    else:
        pad = ((1, 1), (1, 1))
    y = lax.conv_general_dilated(x, jnp.transpose(w, (2, 3, 1, 0)), (1, 1), pad,
                                 dimension_numbers=('NHWC', 'HWIO', 'NHWC'))
    return y + b[None, None, None, :]


def _avgpool2_nhwc(x):
    return lax.reduce_window(x, 0.0, lax.add, (1, 2, 2, 1), (1, 2, 2, 1), 'VALID') / 4.0


def _upsample2_nhwc(x):
    n, h, w, c = x.shape
    x = jnp.broadcast_to(x[:, :, None, :, None, :], (n, h, 2, w, 2, c))
    return x.reshape(n, h * 2, w * 2, c)


def _uniform_grid(b, w, h):
    px = jnp.linspace(-1.0, 1.0, w, dtype=jnp.float32)
    py = jnp.linspace(-1.0, 1.0, h, dtype=jnp.float32)
    gx = jnp.broadcast_to(px[:, None], (w, h))
    gy = jnp.broadcast_to(py[None, :], (w, h))
    g = jnp.stack([gx, gy], axis=0)
    return jnp.broadcast_to(g[None], (b, 2, w, h))


def _sc_body(x_hbm, idx_hbm, out_hbm, idx_v, cur0, new0, cur1, new1):
    wid = lax.axis_index("s") * 2 + lax.axis_index("c")
    pltpu.sync_copy(idx_hbm, idx_v)
    rows = ((cur0, new0), (cur1, new1))
    for rloc in range(2):
        pltpu.sync_copy(x_hbm.at[2 * wid + rloc], rows[rloc][0])

    def step(i, carry):
        scale = jnp.where(i == 0, 128.0, 0.125).astype(jnp.float32)
        for rloc in range(2):
            cur, new = rows[rloc]

            def chunk(j, c2):
                b0 = pl.multiple_of(j * _LANES, _LANES)
                acc = plsc.load_gather(cur, [idx_v[0, pl.ds(b0, _LANES)]])
                for k in range(1, _KNN):
                    acc = acc + plsc.load_gather(cur, [idx_v[k, pl.ds(b0, _LANES)]])
                new[pl.ds(b0, _LANES)] = acc * scale
                return c2

            lax.fori_loop(0, _CHUNKS, chunk, None)
            r = 2 * wid + rloc
            pltpu.sync_copy(new, out_hbm.at[r // _C, i, r % _C])

            def copyback(j, c2):
                b0 = pl.multiple_of(j * _LANES, _LANES)
                cur[pl.ds(b0, _LANES)] = new[pl.ds(b0, _LANES)]
                return c2

            lax.fori_loop(0, _CHUNKS, copyback, None)
        return carry

    lax.fori_loop(0, _NF, step, None)


@jax.jit
def _sc_gather_steps(x_rows, knn_idx):
    mesh = plsc.VectorSubcoreMesh(core_axis_name="c", subcore_axis_name="s")
    f = pl.kernel(
        _sc_body,
        out_type=jax.ShapeDtypeStruct((_B, _NF, _C, _N), jnp.float32),
        mesh=mesh,
        scratch_types=[
            pltpu.VMEM((_KNN, _N), jnp.int32),
            pltpu.VMEM((_N,), jnp.float32),
            pltpu.VMEM((_N,), jnp.float32),
            pltpu.VMEM((_N,), jnp.float32),
            pltpu.VMEM((_N,), jnp.float32),
        ],
        compiler_params=pltpu.CompilerParams(needs_layout_passes=False),
    )
    return f(x_rows, knn_idx)


def kernel(x, conv1_w, conv1_b, ggd_w, ggd_b, conv2_w, conv2_b, vel_w1,
           vel_b1, vel_w2, vel_b2, dec1_w, dec1_b, ggu_w, ggu_b, dec3_w,
           dec3_b):
    x_orig = x
    x = x * 2.0 - 1.0
    xh = jnp.transpose(x, (0, 2, 3, 1))
    h = jax.nn.relu(_conv2d_nhwc(xh, conv1_w, conv1_b, reflect=True))
    h = jax.nn.relu(_conv2d_nhwc(h, ggd_w, ggd_b, reflect=False))
    h = _avgpool2_nhwc(h)
    h = jax.nn.relu(h)
    h = jax.nn.relu(_conv2d_nhwc(h, conv2_w, conv2_b, reflect=True))
    h = _avgpool2_nhwc(h)
    x_part = jnp.transpose(h.reshape(_B, _N, _C), (0, 2, 1))

    pos = _uniform_grid(_B, 32, 32).reshape(_B, 2, _N)
    d = jnp.einsum('bci,bcj->bij', pos, pos)
    _, knn_ind = lax.top_k(-d, _KNN)
    idx = jnp.transpose(knn_ind, (0, 2, 1))[0]

    frames = _sc_gather_steps(x_part.reshape(_ROWS, _N), idx)
    y = jnp.transpose(frames, (0, 1, 3, 2)).reshape(_B * _NF, 32, 32, _C)

    y = _upsample2_nhwc(y)
    y = jax.nn.relu(_conv2d_nhwc(y, dec1_w, dec1_b, reflect=True))
    y = jax.nn.relu(_conv2d_nhwc(y, ggu_w, ggu_b, reflect=False))
    y = _upsample2_nhwc(y)
    y = jnp.tanh(_conv2d_nhwc(y, dec3_w, dec3_b, reflect=True))
    y = jnp.transpose(y, (0, 3, 1, 2)).reshape(_B, _NF, 3, 128, 128)
    y = (y + 1.0) / 2.0
    return jnp.concatenate([x_orig[:, None], y], axis=1)
